# Initial kernel scaffold; baseline (speedup 1.0000x reference)
import functools

import jax
import jax.numpy as jnp
from jax import lax
from jax.experimental import pallas as pl
from jax.experimental.pallas import tpu as pltpu
from jax.experimental.pallas import tpu_sc as plsc

VOCAB = 1000000
EMBED_DIM = 32
BATCH = 4096
HIST_LEN = 200

_NUM_WORKERS = 32
_TOTAL = BATCH * HIST_LEN
_B_PER_W = _TOTAL // _NUM_WORKERS
_CHUNK = 1600
_NCHUNK = _B_PER_W // _CHUNK


@functools.partial(
    pl.kernel,
    mesh=plsc.VectorSubcoreMesh(core_axis_name="c", subcore_axis_name="s"),
    out_type=jax.ShapeDtypeStruct((_TOTAL, EMBED_DIM), jnp.float32),
    scratch_types=[
        pltpu.VMEM((_CHUNK,), jnp.int32),
        pltpu.VMEM((_CHUNK, EMBED_DIM), jnp.float32),
        pltpu.SemaphoreType.DMA,
    ],
)
def _embed_gather(idx_hbm, table_hbm, out_hbm, idx_v, rows_v, sem):
    wid = lax.axis_index("s") * 2 + lax.axis_index("c")
    wbase = wid * _B_PER_W

    def body(g, carry):
        base = wbase + g * _CHUNK
        pltpu.sync_copy(idx_hbm.at[pl.ds(base, _CHUNK)], idx_v)
        pltpu.async_copy(table_hbm.at[idx_v], rows_v, sem).wait()
        pltpu.sync_copy(rows_v, out_hbm.at[pl.ds(base, _CHUNK)])
        return carry

    lax.fori_loop(0, _NCHUNK, body, 0)


def kernel(indices, table):
    flat_idx = indices.reshape(-1).astype(jnp.int32)
    out = _embed_gather(flat_idx, table)
    return out.reshape(BATCH, HIST_LEN, EMBED_DIM)


# speedup vs baseline: 1.4770x; 1.4770x over previous
import functools

import jax
import jax.numpy as jnp
from jax import lax
from jax.experimental import pallas as pl
from jax.experimental.pallas import tpu as pltpu
from jax.experimental.pallas import tpu_sc as plsc

VOCAB = 1000000
EMBED_DIM = 32
BATCH = 4096
HIST_LEN = 200

_NUM_WORKERS = 32
_TOTAL = BATCH * HIST_LEN
_B_PER_W = _TOTAL // _NUM_WORKERS
_CHUNK = 1600
_NCHUNK = _B_PER_W // _CHUNK


@functools.partial(
    pl.kernel,
    mesh=plsc.VectorSubcoreMesh(core_axis_name="c", subcore_axis_name="s"),
    out_type=jax.ShapeDtypeStruct((_TOTAL, EMBED_DIM), jnp.float32),
    scratch_types=[
        pltpu.VMEM((_CHUNK,), jnp.int32),
        pltpu.VMEM((_CHUNK, EMBED_DIM), jnp.float32),
        pltpu.SemaphoreType.DMA,
    ],
    compiler_params=pltpu.CompilerParams(use_tc_tiling_on_sc=False),
)
def _embed_gather(idx_hbm, table_hbm, out_hbm, idx_v, rows_v, sem):
    wid = lax.axis_index("s") * 2 + lax.axis_index("c")
    wbase = wid * _B_PER_W

    def body(g, carry):
        base = wbase + g * _CHUNK
        pltpu.sync_copy(idx_hbm.at[pl.ds(base, _CHUNK)], idx_v)
        pltpu.async_copy(table_hbm.at[idx_v], rows_v, sem).wait()
        pltpu.sync_copy(rows_v, out_hbm.at[pl.ds(base, _CHUNK)])
        return carry

    lax.fori_loop(0, _NCHUNK, body, 0)


def kernel(indices, table):
    flat_idx = indices.reshape(-1).astype(jnp.int32)
    out = _embed_gather(flat_idx, table)
    return out.reshape(BATCH, HIST_LEN, EMBED_DIM)


# speedup vs baseline: 1.5016x; 1.0166x over previous
import functools

import jax
import jax.numpy as jnp
from jax import lax
from jax.experimental import pallas as pl
from jax.experimental.pallas import tpu as pltpu
from jax.experimental.pallas import tpu_sc as plsc

VOCAB = 1000000
EMBED_DIM = 32
BATCH = 4096
HIST_LEN = 200

_NUM_WORKERS = 32
_TOTAL = BATCH * HIST_LEN
_B_PER_W = _TOTAL // _NUM_WORKERS
_CHUNK = 1600
_NCHUNK = _B_PER_W // _CHUNK


@functools.partial(
    pl.kernel,
    mesh=plsc.VectorSubcoreMesh(core_axis_name="c", subcore_axis_name="s"),
    out_type=jax.ShapeDtypeStruct((_TOTAL, EMBED_DIM), jnp.float32),
    scratch_types=[
        pltpu.VMEM((_CHUNK,), jnp.int32),
        pltpu.VMEM((_CHUNK,), jnp.int32),
        pltpu.VMEM((_CHUNK, EMBED_DIM), jnp.float32),
        pltpu.VMEM((_CHUNK, EMBED_DIM), jnp.float32),
        pltpu.SemaphoreType.DMA,
        pltpu.SemaphoreType.DMA,
        pltpu.SemaphoreType.DMA,
        pltpu.SemaphoreType.DMA,
        pltpu.SemaphoreType.DMA,
        pltpu.SemaphoreType.DMA,
    ],
    compiler_params=pltpu.CompilerParams(use_tc_tiling_on_sc=False),
)
def _embed_gather(idx_hbm, table_hbm, out_hbm,
                  idx0, idx1, rows0, rows1, si0, si1, sg0, sg1, ss0, ss1):
    idx_v = (idx0, idx1)
    rows_v = (rows0, rows1)
    si = (si0, si1)
    sg = (sg0, sg1)
    ss = (ss0, ss1)

    wid = lax.axis_index("s") * 2 + lax.axis_index("c")
    wbase = wid * _B_PER_W

    def idx_slice(c):
        return idx_hbm.at[pl.ds(wbase + c * _CHUNK, _CHUNK)]

    def out_slice(c):
        return out_hbm.at[pl.ds(wbase + c * _CHUNK, _CHUNK)]

    def start_idx(c, b):
        pltpu.async_copy(idx_slice(c), idx_v[b], si[b])

    def wait_idx(b):
        pltpu.make_async_copy(idx_slice(0), idx_v[b], si[b]).wait()

    def start_gather(b):
        pltpu.async_copy(table_hbm.at[idx_v[b]], rows_v[b], sg[b])

    def wait_gather(b):
        pltpu.make_async_copy(table_hbm.at[idx_v[b]], rows_v[b], sg[b]).wait()

    def start_store(c, b):
        pltpu.async_copy(rows_v[b], out_slice(c), ss[b])

    def wait_store(b):
        pltpu.make_async_copy(rows_v[b], out_slice(0), ss[b]).wait()

    start_idx(0, 0)
    wait_idx(0)
    start_gather(0)
    start_idx(1, 1)
    wait_idx(1)
    start_gather(1)
    wait_gather(0)
    start_idx(2, 0)
    start_store(0, 0)

    def body(g, carry):
        for b in range(2):
            c = 2 * g + b
            ob = 1 - b
            wait_store(b)
            wait_idx(b)
            start_gather(b)
            wait_gather(ob)
            start_idx((c + 1) % _NCHUNK, ob)
            start_store(c - 1, ob)
        return carry

    lax.fori_loop(1, _NCHUNK // 2, body, 0)

    wait_gather(1)
    start_store(_NCHUNK - 1, 1)
    wait_idx(0)
    wait_store(0)
    wait_store(1)


def kernel(indices, table):
    flat_idx = indices.reshape(-1).astype(jnp.int32)
    out = _embed_gather(flat_idx, table)
    return out.reshape(BATCH, HIST_LEN, EMBED_DIM)
